# baseline (speedup 1.0000x reference)
import functools

import jax
import jax.numpy as jnp
from jax import lax
from jax.experimental import pallas as pl
from jax.experimental.pallas import tpu as pltpu
from jax.experimental.pallas import tpu_sc as plsc

NUM_EMB = 1000000
EMB_DIM = 64
SUB_DIM = 32
BASE_SEED = 42
N = 4096 * 26
NW = 32
NPW = N // NW
CHUNK = 128
NCH = NPW // CHUNK
VPC = CHUNK // 16

_MAGIC_HI = 1125899907 >> 16
_MAGIC_LO = 1125899907 & 0xFFFF

_MESH = plsc.VectorSubcoreMesh(core_axis_name="c", subcore_axis_name="s")
_PARAMS = pltpu.CompilerParams(use_tc_tiling_on_sc=False)


def _umod_1e6(h):
    al = h & jnp.uint32(0xFFFF)
    ah = h >> 16
    t = ah * jnp.uint32(_MAGIC_LO) + ((al * jnp.uint32(_MAGIC_LO)) >> 16)
    t2 = al * jnp.uint32(_MAGIC_HI) + (t & jnp.uint32(0xFFFF))
    hi = ah * jnp.uint32(_MAGIC_HI) + (t >> 16) + (t2 >> 16)
    q = hi >> 18
    return h - q * jnp.uint32(NUM_EMB)


def _hash16(v_u32, seed):
    h = v_u32 ^ jnp.uint32(seed)
    h = h * jnp.uint32(2654435761)
    h = h ^ (h >> 16)
    h = h * jnp.uint32(2246822519)
    h = h ^ (h >> 13)
    return plsc.bitcast(_umod_1e6(h), jnp.int32)


def _hash_body(ids_hbm, idx0_hbm, idx1_hbm, ids_v, idx0_v, idx1_v):
    wid = lax.axis_index("s") * 2 + lax.axis_index("c")
    base = wid * NPW
    pltpu.sync_copy(ids_hbm.at[pl.ds(base, NPW)], ids_v)

    def hash_chunk(c):
        for j in range(VPC):
            u = plsc.bitcast(ids_v[pl.ds(c * CHUNK + j * 16, 16)], jnp.uint32)
            idx0_v[c, pl.ds(j * 16, 16)] = _hash16(u, BASE_SEED)
            idx1_v[c, pl.ds(j * 16, 16)] = _hash16(u, BASE_SEED + 1)

    lax.fori_loop(0, NCH, lambda c, _: (hash_chunk(c), 0)[1], 0)
    pltpu.sync_copy(idx0_v, idx0_hbm.at[wid])
    pltpu.sync_copy(idx1_v, idx1_hbm.at[wid])


_hash_k = functools.partial(
    pl.kernel,
    out_type=(jax.ShapeDtypeStruct((NW, NCH, CHUNK), jnp.int32),
              jax.ShapeDtypeStruct((NW, NCH, CHUNK), jnp.int32)),
    mesh=_MESH,
    compiler_params=_PARAMS,
    scratch_types=[
        pltpu.VMEM((NPW,), jnp.int32),
        pltpu.VMEM((NCH, CHUNK), jnp.int32),
        pltpu.VMEM((NCH, CHUNK), jnp.int32),
    ],
)(_hash_body)


def _g0_body(idx0_hbm, w0_hbm, outa_hbm, idx0_v, buf, sem):
    wid = lax.axis_index("s") * 2 + lax.axis_index("c")
    base = wid * NPW
    pltpu.sync_copy(idx0_hbm.at[wid], idx0_v)

    def gather_chunk(c, _):
        pltpu.async_copy(w0_hbm.at[idx0_v.at[c]], buf, sem).wait()
        pltpu.sync_copy(buf, outa_hbm.at[pl.ds(base + c * CHUNK, CHUNK)])
        return 0

    lax.fori_loop(0, NCH, gather_chunk, 0)


_g0_k = functools.partial(
    pl.kernel,
    out_type=jax.ShapeDtypeStruct((N, SUB_DIM), jnp.float32),
    mesh=_MESH,
    compiler_params=_PARAMS,
    scratch_types=[
        pltpu.VMEM((NCH, CHUNK), jnp.int32),
        pltpu.VMEM((CHUNK, SUB_DIM), jnp.float32),
        pltpu.SemaphoreType.DMA,
    ],
)(_g0_body)


def _g1_body(idx1_hbm, w1_hbm, outa_hbm, out_hbm,
             idx1_v, buf0, buf1, sem0, sem1):
    wid = lax.axis_index("s") * 2 + lax.axis_index("c")
    base = wid * NPW
    pltpu.sync_copy(idx1_hbm.at[wid], idx1_v)

    def gather_chunk(c, _):
        row = base + c * CHUNK
        cp1 = pltpu.async_copy(w1_hbm.at[idx1_v.at[c]], buf1, sem1)
        cp0 = pltpu.async_copy(outa_hbm.at[pl.ds(row, CHUNK)], buf0, sem0)
        cp0.wait()
        cp1.wait()
        pltpu.sync_copy(buf0, out_hbm.at[pl.ds(row, CHUNK), pl.ds(0, SUB_DIM)])
        pltpu.sync_copy(buf1, out_hbm.at[pl.ds(row, CHUNK), pl.ds(SUB_DIM, SUB_DIM)])
        return 0

    lax.fori_loop(0, NCH, gather_chunk, 0)


_g1_k = functools.partial(
    pl.kernel,
    out_type=jax.ShapeDtypeStruct((N, EMB_DIM), jnp.float32),
    mesh=_MESH,
    compiler_params=_PARAMS,
    scratch_types=[
        pltpu.VMEM((NCH, CHUNK), jnp.int32),
        pltpu.VMEM((CHUNK, SUB_DIM), jnp.float32),
        pltpu.VMEM((CHUNK, SUB_DIM), jnp.float32),
        pltpu.SemaphoreType.DMA,
        pltpu.SemaphoreType.DMA,
    ],
)(_g1_body)


@jax.jit
def kernel(input_ids, W0, W1):
    flat = input_ids.reshape(-1)
    idx0, idx1 = _hash_k(flat)
    outa = _g0_k(idx0, W0)
    out = _g1_k(idx1, W1, outa)
    return out.reshape(input_ids.shape + (EMB_DIM,))
